# baseline (speedup 1.0000x reference)
import functools

import jax
import jax.numpy as jnp
from jax import lax
from jax.experimental import pallas as pl
from jax.experimental.pallas import tpu as pltpu
from jax.experimental.pallas import tpu_sc as plsc

B = 16
T = 2048
D = 1024


def _take_last_sc(xf, seq_len_i32):
    mesh = plsc.VectorSubcoreMesh(core_axis_name="c", subcore_axis_name="s")

    @functools.partial(
        pl.kernel,
        mesh=mesh,
        out_type=jax.ShapeDtypeStruct((B, D), jnp.float32),
        scratch_types=[
            pltpu.VMEM((B,), jnp.int32),
            pltpu.VMEM((B, D), jnp.float32),
            pltpu.SemaphoreType.DMA,
        ],
    )
    def k(table_hbm, len_hbm, out_hbm, idx_v, rows_v, sem):
        wid = lax.axis_index("s") * 2 + lax.axis_index("c")

        @pl.when(wid == 0)
        def _():
            pltpu.sync_copy(len_hbm, idx_v)
            idx_v[...] = lax.iota(jnp.int32, B) * T + idx_v[...] - 1
            pltpu.async_copy(table_hbm.at[idx_v], rows_v, sem).wait()
            pltpu.sync_copy(rows_v, out_hbm)

    return k(xf, seq_len_i32)


def kernel(x, seq_len):
    xf = x.reshape(B * T, D)
    out = _take_last_sc(xf, seq_len.astype(jnp.int32))
    return (out, None)


# speedup vs baseline: 1.0769x; 1.0769x over previous
import functools

import jax
import jax.numpy as jnp
from jax import lax
from jax.experimental import pallas as pl
from jax.experimental.pallas import tpu as pltpu
from jax.experimental.pallas import tpu_sc as plsc

B = 16
T = 2048
D = 1024


def _take_last_sc(xf, seq_len_i32):
    mesh = plsc.VectorSubcoreMesh(core_axis_name="c", subcore_axis_name="s",
                                  num_cores=1)

    @functools.partial(
        pl.kernel,
        mesh=mesh,
        out_type=jax.ShapeDtypeStruct((B, D), jnp.float32),
        scratch_types=[
            pltpu.VMEM((B,), jnp.int32),
            pltpu.VMEM((B, D), jnp.float32),
            pltpu.SemaphoreType.DMA,
        ],
    )
    def k(table_hbm, len_hbm, out_hbm, idx_v, rows_v, sem):
        wid = lax.axis_index("s")

        @pl.when(wid == 0)
        def _():
            pltpu.sync_copy(len_hbm, idx_v)
            idx_v[...] = lax.iota(jnp.int32, B) * T + idx_v[...] - 1
            pltpu.async_copy(table_hbm.at[idx_v], rows_v, sem).wait()
            pltpu.sync_copy(rows_v, out_hbm)

    return k(xf, seq_len_i32)


def kernel(x, seq_len):
    xf = x.reshape(B * T, D)
    out = _take_last_sc(xf, seq_len.astype(jnp.int32))
    return (out, None)


# speedup vs baseline: 1.1128x; 1.0334x over previous
import functools

import jax
import jax.numpy as jnp
from jax import lax
from jax.experimental import pallas as pl
from jax.experimental.pallas import tpu as pltpu
from jax.experimental.pallas import tpu_sc as plsc

B = 16
T = 2048
D = 1024


def _take_last_sc(xf, seq_len_i32):
    mesh = plsc.ScalarSubcoreMesh(axis_name="c", num_cores=1)

    @functools.partial(
        pl.kernel,
        mesh=mesh,
        out_type=jax.ShapeDtypeStruct((B, D), jnp.float32),
        scratch_types=[
            pltpu.SMEM((B,), jnp.int32),
            pltpu.SemaphoreType.DMA,
        ],
    )
    def k(x_hbm, len_hbm, out_hbm, len_s, sem):
        pltpu.sync_copy(len_hbm, len_s)
        copies = [
            pltpu.async_copy(x_hbm.at[b, len_s[b] - 1], out_hbm.at[b], sem)
            for b in range(B)
        ]
        for cp in copies:
            cp.wait()

    return k(xf, seq_len_i32)


def kernel(x, seq_len):
    out = _take_last_sc(x, seq_len.astype(jnp.int32))
    return (out, None)


# speedup vs baseline: 1.1279x; 1.0136x over previous
import functools

import jax
import jax.numpy as jnp
from jax import lax
from jax.experimental import pallas as pl
from jax.experimental.pallas import tpu as pltpu
from jax.experimental.pallas import tpu_sc as plsc

B = 16
T = 2048
D = 1024


def _take_last_sc(xf, seq_len_i32):
    mesh = plsc.ScalarSubcoreMesh(axis_name="c", num_cores=1)

    @functools.partial(
        pl.kernel,
        mesh=mesh,
        out_type=jax.ShapeDtypeStruct((B, D), jnp.float32),
        scratch_types=[
            pltpu.SMEM((B,), jnp.int32),
            pltpu.SemaphoreType.DMA,
        ],
    )
    def k(x_hbm, len_hbm, out_hbm, len_s, sem):
        pltpu.sync_copy(len_hbm, len_s)

        def body(b, carry):
            pltpu.async_copy(x_hbm.at[b, len_s[b] - 1], out_hbm.at[b], sem)
            return carry

        lax.fori_loop(0, B, body, 0)
        pltpu.make_async_copy(x_hbm.at[0, pl.ds(0, B)], out_hbm, sem).wait()

    return k(xf, seq_len_i32)


def kernel(x, seq_len):
    out = _take_last_sc(x, seq_len.astype(jnp.int32))
    return (out, None)
